# Initial kernel scaffold; baseline (speedup 1.0000x reference)
import jax
import jax.numpy as jnp
from jax.experimental import pallas as pl


def kernel(features, features0, edge_index, edge_weight):
    raise NotImplementedError("write your pallas kernel here")


# speedup vs baseline: 4.4875x; 4.4875x over previous
import functools

import jax
import jax.numpy as jnp
from jax import lax
from jax.experimental import pallas as pl
from jax.experimental.pallas import tpu as pltpu
from jax.experimental.pallas import tpu_sc as plsc

_ALPHA = 0.1
_LANES = 16


def _sc_partials(features, src, dst, weight):
    n, d = features.shape
    e = weight.shape[0]
    info = plsc.get_sparse_core_info()
    nc, ns = info.num_cores, info.num_subcores
    nw = nc * ns

    per_tile = e // nw
    assert per_tile * nw == e
    chunk = 8
    for c in range(128, 7, -8):
        if per_tile % c == 0:
            chunk = c
            break
    n_chunks = per_tile // chunk

    assert n % chunk == 0
    n_row_chunks = n // chunk

    mesh = plsc.VectorSubcoreMesh(core_axis_name="c", subcore_axis_name="s")

    @functools.partial(
        pl.kernel,
        mesh=mesh,
        out_type=jax.ShapeDtypeStruct((nc, n, d), jnp.float32),
        scratch_types=[
            pltpu.VMEM_SHARED((n, d), jnp.float32),
            pltpu.VMEM((chunk,), jnp.int32),
            pltpu.VMEM((chunk,), jnp.int32),
            pltpu.VMEM((chunk,), jnp.float32),
            pltpu.VMEM((chunk, d), jnp.float32),
            pltpu.SemaphoreType.DMA,
        ],
    )
    def sc_kernel(feat_hbm, src_hbm, dst_hbm, w_hbm, out_hbm,
                  hi_sh, src_v, dst_v, w_v, rows_v, sem):
        cid = lax.axis_index("c")
        sid = lax.axis_index("s")
        wid = sid * nc + cid

        def zero_row(r, carry):
            for j in range(d // _LANES):
                rows_v[r, pl.ds(j * _LANES, _LANES)] = jnp.zeros(
                    (_LANES,), jnp.float32)
            return carry
        lax.fori_loop(0, chunk, zero_row, 0)

        def zero_chunk(k, carry):
            rc = sid + ns * k

            @pl.when(rc < n_row_chunks)
            def _():
                pltpu.sync_copy(rows_v, hi_sh.at[pl.ds(rc * chunk, chunk)])
            return carry
        lax.fori_loop(0, (n_row_chunks + ns - 1) // ns, zero_chunk, 0)
        plsc.subcore_barrier()

        edge0 = wid * per_tile

        def chunk_body(k, carry):
            base = edge0 + k * chunk
            pltpu.sync_copy(src_hbm.at[pl.ds(base, chunk)], src_v)
            pltpu.sync_copy(dst_hbm.at[pl.ds(base, chunk)], dst_v)
            pltpu.sync_copy(w_hbm.at[pl.ds(base, chunk)], w_v)
            pltpu.async_copy(feat_hbm.at[src_v], rows_v, sem).wait()

            def scale_group(g, c2):
                wvec = w_v[pl.ds(g * _LANES, _LANES)]
                for i in range(_LANES):
                    r = g * _LANES + i
                    wspl = jnp.full((_LANES,), wvec[i], jnp.float32)
                    for j in range(d // _LANES):
                        sl = pl.ds(j * _LANES, _LANES)
                        rows_v[r, sl] = rows_v[r, sl] * wspl
                return c2
            lax.fori_loop(0, chunk // _LANES, scale_group, 0)

            pltpu.sync_copy(rows_v, hi_sh.at[dst_v], add=True)
            return carry
        lax.fori_loop(0, n_chunks, chunk_body, 0)

        plsc.subcore_barrier()

        def out_chunk(k, carry):
            rc = sid + ns * k

            @pl.when(rc < n_row_chunks)
            def _():
                pltpu.sync_copy(hi_sh.at[pl.ds(rc * chunk, chunk)],
                                out_hbm.at[cid, pl.ds(rc * chunk, chunk)])
            return carry
        lax.fori_loop(0, (n_row_chunks + ns - 1) // ns, out_chunk, 0)

    return sc_kernel(features, src, dst, weight)


def _combine(p0, p1, features0):
    n, d = features0.shape
    blk = 2000
    assert n % blk == 0

    def body(p0_ref, p1_ref, f0_ref, o_ref):
        hi = p0_ref[...] + p1_ref[...]
        x = jnp.float32(1.0 - _ALPHA) * hi + jnp.float32(_ALPHA) * f0_ref[...]
        o_ref[...] = jnp.maximum(x, jnp.float32(0.0))

    spec = pl.BlockSpec((blk, d), lambda i: (i, 0))
    return pl.pallas_call(
        body,
        grid=(n // blk,),
        in_specs=[spec, spec, spec],
        out_specs=spec,
        out_shape=jax.ShapeDtypeStruct((n, d), jnp.float32),
    )(p0, p1, features0)


@jax.jit
def kernel(features, features0, edge_index, edge_weight):
    dst = edge_index[0]
    src = edge_index[1]
    partials = _sc_partials(features, src, dst, edge_weight)
    return _combine(partials[0], partials[1], features0)


# speedup vs baseline: 6.2039x; 1.3825x over previous
import functools

import jax
import jax.numpy as jnp
from jax import lax
from jax.experimental import pallas as pl
from jax.experimental.pallas import tpu as pltpu
from jax.experimental.pallas import tpu_sc as plsc

_ALPHA = 0.1
_LANES = 16


def _sc_partials(features, src, dst, weight):
    n, d = features.shape
    e = weight.shape[0]
    info = plsc.get_sparse_core_info()
    nc, ns = info.num_cores, info.num_subcores
    nw = nc * ns

    per_tile = e // nw
    assert per_tile * nw == e
    chunk = 8
    for c in range(128, 7, -8):
        if per_tile % c == 0:
            chunk = c
            break
    n_chunks = per_tile // chunk

    assert n % chunk == 0
    n_row_chunks = n // chunk

    mesh = plsc.VectorSubcoreMesh(core_axis_name="c", subcore_axis_name="s")

    @functools.partial(
        pl.kernel,
        mesh=mesh,
        out_type=jax.ShapeDtypeStruct((nc, n, d), jnp.float32),
        scratch_types=[
            pltpu.VMEM_SHARED((n, d), jnp.float32),
            pltpu.VMEM((per_tile,), jnp.int32),
            pltpu.VMEM((per_tile,), jnp.int32),
            pltpu.VMEM((per_tile,), jnp.float32),
            pltpu.VMEM((chunk, d), jnp.float32),
            pltpu.SemaphoreType.DMA,
        ],
    )
    def sc_kernel(feat_hbm, src_hbm, dst_hbm, w_hbm, out_hbm,
                  hi_sh, src_v, dst_v, w_v, rows_v, sem):
        cid = lax.axis_index("c")
        sid = lax.axis_index("s")
        wid = sid * nc + cid

        def zero_row(r, carry):
            for j in range(d // _LANES):
                rows_v[r, pl.ds(j * _LANES, _LANES)] = jnp.zeros(
                    (_LANES,), jnp.float32)
            return carry
        lax.fori_loop(0, chunk, zero_row, 0)

        def zero_chunk(k, carry):
            rc = sid + ns * k

            @pl.when(rc < n_row_chunks)
            def _():
                pltpu.sync_copy(rows_v, hi_sh.at[pl.ds(rc * chunk, chunk)])
            return carry
        lax.fori_loop(0, (n_row_chunks + ns - 1) // ns, zero_chunk, 0)
        plsc.subcore_barrier()

        edge0 = wid * per_tile
        pltpu.sync_copy(src_hbm.at[pl.ds(edge0, per_tile)], src_v)
        pltpu.sync_copy(dst_hbm.at[pl.ds(edge0, per_tile)], dst_v)
        pltpu.sync_copy(w_hbm.at[pl.ds(edge0, per_tile)], w_v)

        def chunk_body(k, carry):
            base = k * chunk
            pltpu.async_copy(
                feat_hbm.at[src_v.at[pl.ds(base, chunk)]], rows_v, sem).wait()

            def scale_group(g, c2):
                wvec = w_v[pl.ds(base + g * _LANES, _LANES)]
                for i in range(_LANES):
                    r = g * _LANES + i
                    wspl = jnp.full((_LANES,), wvec[i], jnp.float32)
                    for j in range(d // _LANES):
                        sl = pl.ds(j * _LANES, _LANES)
                        rows_v[r, sl] = rows_v[r, sl] * wspl
                return c2
            lax.fori_loop(0, chunk // _LANES, scale_group, 0)

            for g in range(chunk // _LANES):
                dst16 = dst_v[pl.ds(base + g * _LANES, _LANES)]
                pltpu.sync_copy(rows_v.at[pl.ds(g * _LANES, _LANES)],
                                hi_sh.at[dst16], add=True)
            return carry
        lax.fori_loop(0, n_chunks, chunk_body, 0)

        plsc.subcore_barrier()

        def out_chunk(k, carry):
            rc = sid + ns * k

            @pl.when(rc < n_row_chunks)
            def _():
                pltpu.sync_copy(hi_sh.at[pl.ds(rc * chunk, chunk)],
                                out_hbm.at[cid, pl.ds(rc * chunk, chunk)])
            return carry
        lax.fori_loop(0, (n_row_chunks + ns - 1) // ns, out_chunk, 0)

    return sc_kernel(features, src, dst, weight)


def _combine(p0, p1, features0):
    n, d = features0.shape
    blk = 2000
    assert n % blk == 0

    def body(p0_ref, p1_ref, f0_ref, o_ref):
        hi = p0_ref[...] + p1_ref[...]
        x = jnp.float32(1.0 - _ALPHA) * hi + jnp.float32(_ALPHA) * f0_ref[...]
        o_ref[...] = jnp.maximum(x, jnp.float32(0.0))

    spec = pl.BlockSpec((blk, d), lambda i: (i, 0))
    return pl.pallas_call(
        body,
        grid=(n // blk,),
        in_specs=[spec, spec, spec],
        out_specs=spec,
        out_shape=jax.ShapeDtypeStruct((n, d), jnp.float32),
    )(p0, p1, features0)


@jax.jit
def kernel(features, features0, edge_index, edge_weight):
    dst = edge_index[0]
    src = edge_index[1]
    partials = _sc_partials(features, src, dst, edge_weight)
    return _combine(partials[0], partials[1], features0)


# speedup vs baseline: 10.1471x; 1.6356x over previous
import functools

import jax
import jax.numpy as jnp
from jax import lax
from jax.experimental import pallas as pl
from jax.experimental.pallas import tpu as pltpu
from jax.experimental.pallas import tpu_sc as plsc

_ALPHA = 0.1
_LANES = 16


def _sc_partials(features, src, dst, weight):
    n, d = features.shape
    e = weight.shape[0]
    info = plsc.get_sparse_core_info()
    nc, ns = info.num_cores, info.num_subcores
    nw = nc * ns

    per_tile = e // nw
    assert per_tile * nw == e
    chunk = 8
    for c in range(128, 7, -8):
        if per_tile % c == 0:
            chunk = c
            break
    n_chunks = per_tile // chunk

    assert n % chunk == 0
    n_row_chunks = n // chunk

    mesh = plsc.VectorSubcoreMesh(core_axis_name="c", subcore_axis_name="s")

    @functools.partial(
        pl.kernel,
        mesh=mesh,
        out_type=jax.ShapeDtypeStruct((nc, n, d), jnp.float32),
        scratch_types=[
            pltpu.VMEM_SHARED((n, d), jnp.float32),
            pltpu.VMEM((per_tile,), jnp.int32),
            pltpu.VMEM((per_tile,), jnp.int32),
            pltpu.VMEM((per_tile,), jnp.float32),
            pltpu.VMEM((chunk, d), jnp.float32),
            pltpu.VMEM((chunk, d), jnp.float32),
            pltpu.SemaphoreType.DMA,
            pltpu.SemaphoreType.DMA,
        ],
    )
    def sc_kernel(feat_hbm, src_hbm, dst_hbm, w_hbm, out_hbm,
                  hi_sh, src_v, dst_v, w_v, rows_a, rows_b, sem_a, sem_b):
        rows_v = rows_a
        cid = lax.axis_index("c")
        sid = lax.axis_index("s")
        wid = sid * nc + cid

        def zero_row(r, carry):
            for j in range(d // _LANES):
                rows_v[r, pl.ds(j * _LANES, _LANES)] = jnp.zeros(
                    (_LANES,), jnp.float32)
            return carry
        lax.fori_loop(0, chunk, zero_row, 0)

        def zero_chunk(k, carry):
            rc = sid + ns * k

            @pl.when(rc < n_row_chunks)
            def _():
                pltpu.sync_copy(rows_v, hi_sh.at[pl.ds(rc * chunk, chunk)])
            return carry
        lax.fori_loop(0, (n_row_chunks + ns - 1) // ns, zero_chunk, 0)
        plsc.subcore_barrier()

        edge0 = wid * per_tile
        pltpu.sync_copy(src_hbm.at[pl.ds(edge0, per_tile)], src_v)
        pltpu.sync_copy(dst_hbm.at[pl.ds(edge0, per_tile)], dst_v)
        pltpu.sync_copy(w_hbm.at[pl.ds(edge0, per_tile)], w_v)

        def start_gather(c, buf, sem):
            pltpu.async_copy(
                feat_hbm.at[src_v.at[pl.ds(c * chunk, chunk)]], buf, sem)

        def wait_gather(c, buf, sem):
            pltpu.make_async_copy(
                feat_hbm.at[src_v.at[pl.ds(c * chunk, chunk)]],
                buf, sem).wait()

        def process(c, buf):
            base = c * chunk

            def scale_group(g, c2):
                wvec = w_v[pl.ds(base + g * _LANES, _LANES)]
                for i in range(_LANES):
                    r = g * _LANES + i
                    wspl = jnp.full((_LANES,), wvec[i], jnp.float32)
                    for j in range(d // _LANES):
                        sl = pl.ds(j * _LANES, _LANES)
                        buf[r, sl] = buf[r, sl] * wspl
                return c2
            lax.fori_loop(0, chunk // _LANES, scale_group, 0)

            for g in range(chunk // _LANES):
                dst16 = dst_v[pl.ds(base + g * _LANES, _LANES)]
                pltpu.sync_copy(buf.at[pl.ds(g * _LANES, _LANES)],
                                hi_sh.at[dst16], add=True)

        start_gather(0, rows_a, sem_a)
        npairs = (n_chunks - 1) // 2

        def pair_body(k, carry):
            c0 = 2 * k
            c1 = c0 + 1
            start_gather(c1, rows_b, sem_b)
            wait_gather(c0, rows_a, sem_a)
            process(c0, rows_a)
            start_gather(c0 + 2, rows_a, sem_a)
            wait_gather(c1, rows_b, sem_b)
            process(c1, rows_b)
            return carry
        lax.fori_loop(0, npairs, pair_body, 0)

        t0 = 2 * npairs
        if n_chunks - t0 == 2:
            start_gather(t0 + 1, rows_b, sem_b)
        wait_gather(t0, rows_a, sem_a)
        process(t0, rows_a)
        if n_chunks - t0 == 2:
            wait_gather(t0 + 1, rows_b, sem_b)
            process(t0 + 1, rows_b)

        plsc.subcore_barrier()

        def out_chunk(k, carry):
            rc = sid + ns * k

            @pl.when(rc < n_row_chunks)
            def _():
                pltpu.sync_copy(hi_sh.at[pl.ds(rc * chunk, chunk)],
                                out_hbm.at[cid, pl.ds(rc * chunk, chunk)])
            return carry
        lax.fori_loop(0, (n_row_chunks + ns - 1) // ns, out_chunk, 0)

    return sc_kernel(features, src, dst, weight)


def _combine(p0, p1, features0):
    n, d = features0.shape
    blk = 2000
    assert n % blk == 0

    def body(p0_ref, p1_ref, f0_ref, o_ref):
        hi = p0_ref[...] + p1_ref[...]
        x = jnp.float32(1.0 - _ALPHA) * hi + jnp.float32(_ALPHA) * f0_ref[...]
        o_ref[...] = jnp.maximum(x, jnp.float32(0.0))

    spec = pl.BlockSpec((blk, d), lambda i: (i, 0))
    return pl.pallas_call(
        body,
        grid=(n // blk,),
        in_specs=[spec, spec, spec],
        out_specs=spec,
        out_shape=jax.ShapeDtypeStruct((n, d), jnp.float32),
    )(p0, p1, features0)


@jax.jit
def kernel(features, features0, edge_index, edge_weight):
    dst = edge_index[0]
    src = edge_index[1]
    partials = _sc_partials(features, src, dst, edge_weight)
    return _combine(partials[0], partials[1], features0)


# speedup vs baseline: 11.5784x; 1.1411x over previous
import functools

import jax
import jax.numpy as jnp
from jax import lax
from jax.experimental import pallas as pl
from jax.experimental.pallas import tpu as pltpu
from jax.experimental.pallas import tpu_sc as plsc

_ALPHA = 0.1
_LANES = 16


def _sc_partials(features, src, dst, weight):
    n, d = features.shape
    e = weight.shape[0]
    info = plsc.get_sparse_core_info()
    nc, ns = info.num_cores, info.num_subcores
    nw = nc * ns

    per_tile = e // nw
    assert per_tile * nw == e
    chunk = 8
    for c in range(128, 7, -8):
        if per_tile % c == 0:
            chunk = c
            break
    n_chunks = per_tile // chunk

    assert n % chunk == 0
    n_row_chunks = n // chunk

    mesh = plsc.VectorSubcoreMesh(core_axis_name="c", subcore_axis_name="s")

    @functools.partial(
        pl.kernel,
        mesh=mesh,
        out_type=jax.ShapeDtypeStruct((nc, n, d), jnp.float32),
        scratch_types=[
            pltpu.VMEM_SHARED((n, d), jnp.float32),
            pltpu.VMEM((per_tile,), jnp.int32),
            pltpu.VMEM((per_tile,), jnp.int32),
            pltpu.VMEM((per_tile,), jnp.float32),
            pltpu.VMEM((chunk, d), jnp.float32),
            pltpu.VMEM((chunk, d), jnp.float32),
            pltpu.SemaphoreType.DMA,
            pltpu.SemaphoreType.DMA,
            pltpu.SemaphoreType.DMA,
        ],
    )
    def sc_kernel(feat_hbm, src_hbm, dst_hbm, w_hbm, out_hbm,
                  hi_sh, src_v, dst_v, w_v, rows_a, rows_b,
                  sem_a, sem_b, sem_s):
        rows_v = rows_a
        cid = lax.axis_index("c")
        sid = lax.axis_index("s")
        wid = sid * nc + cid

        def zero_row(r, carry):
            for j in range(d // _LANES):
                rows_v[r, pl.ds(j * _LANES, _LANES)] = jnp.zeros(
                    (_LANES,), jnp.float32)
            return carry
        lax.fori_loop(0, chunk, zero_row, 0)

        def zero_chunk(k, carry):
            rc = sid + ns * k

            @pl.when(rc < n_row_chunks)
            def _():
                pltpu.sync_copy(rows_v, hi_sh.at[pl.ds(rc * chunk, chunk)])
            return carry
        lax.fori_loop(0, (n_row_chunks + ns - 1) // ns, zero_chunk, 0)
        plsc.subcore_barrier()

        edge0 = wid * per_tile
        pltpu.sync_copy(src_hbm.at[pl.ds(edge0, per_tile)], src_v)
        pltpu.sync_copy(dst_hbm.at[pl.ds(edge0, per_tile)], dst_v)
        pltpu.sync_copy(w_hbm.at[pl.ds(edge0, per_tile)], w_v)

        def start_gather(c, buf, sem):
            pltpu.async_copy(
                feat_hbm.at[src_v.at[pl.ds(c * chunk, chunk)]], buf, sem)

        def wait_gather(c, buf, sem):
            pltpu.make_async_copy(
                feat_hbm.at[src_v.at[pl.ds(c * chunk, chunk)]],
                buf, sem).wait()

        def process(c, buf):
            base = c * chunk

            def scale_scatter_group(g, c2):
                wvec = w_v[pl.ds(base + g * _LANES, _LANES)]
                for i in range(_LANES):
                    r = g * _LANES + i
                    wspl = jnp.full((_LANES,), wvec[i], jnp.float32)
                    for j in range(d // _LANES):
                        sl = pl.ds(j * _LANES, _LANES)
                        buf[r, sl] = buf[r, sl] * wspl
                dst16 = dst_v[pl.ds(base + g * _LANES, _LANES)]
                pltpu.async_copy(buf.at[pl.ds(g * _LANES, _LANES)],
                                 hi_sh.at[dst16], sem_s, add=True)
                return c2
            lax.fori_loop(0, chunk // _LANES, scale_scatter_group, 0)

            def drain_group(g, c2):
                dst16 = dst_v[pl.ds(base + g * _LANES, _LANES)]
                pltpu.make_async_copy(buf.at[pl.ds(g * _LANES, _LANES)],
                                      hi_sh.at[dst16], sem_s).wait()
                return c2
            lax.fori_loop(0, chunk // _LANES, drain_group, 0)

        start_gather(0, rows_a, sem_a)
        npairs = (n_chunks - 1) // 2

        def pair_body(k, carry):
            c0 = 2 * k
            c1 = c0 + 1
            start_gather(c1, rows_b, sem_b)
            wait_gather(c0, rows_a, sem_a)
            process(c0, rows_a)
            start_gather(c0 + 2, rows_a, sem_a)
            wait_gather(c1, rows_b, sem_b)
            process(c1, rows_b)
            return carry
        lax.fori_loop(0, npairs, pair_body, 0)

        t0 = 2 * npairs
        if n_chunks - t0 == 2:
            start_gather(t0 + 1, rows_b, sem_b)
        wait_gather(t0, rows_a, sem_a)
        process(t0, rows_a)
        if n_chunks - t0 == 2:
            wait_gather(t0 + 1, rows_b, sem_b)
            process(t0 + 1, rows_b)

        plsc.subcore_barrier()

        def out_chunk(k, carry):
            rc = sid + ns * k

            @pl.when(rc < n_row_chunks)
            def _():
                pltpu.sync_copy(hi_sh.at[pl.ds(rc * chunk, chunk)],
                                out_hbm.at[cid, pl.ds(rc * chunk, chunk)])
            return carry
        lax.fori_loop(0, (n_row_chunks + ns - 1) // ns, out_chunk, 0)

    return sc_kernel(features, src, dst, weight)


def _combine(p0, p1, features0):
    n, d = features0.shape
    blk = 2000
    assert n % blk == 0

    def body(p0_ref, p1_ref, f0_ref, o_ref):
        hi = p0_ref[...] + p1_ref[...]
        x = jnp.float32(1.0 - _ALPHA) * hi + jnp.float32(_ALPHA) * f0_ref[...]
        o_ref[...] = jnp.maximum(x, jnp.float32(0.0))

    spec = pl.BlockSpec((blk, d), lambda i: (i, 0))
    return pl.pallas_call(
        body,
        grid=(n // blk,),
        in_specs=[spec, spec, spec],
        out_specs=spec,
        out_shape=jax.ShapeDtypeStruct((n, d), jnp.float32),
    )(p0, p1, features0)


@jax.jit
def kernel(features, features0, edge_index, edge_weight):
    dst = edge_index[0]
    src = edge_index[1]
    partials = _sc_partials(features, src, dst, edge_weight)
    return _combine(partials[0], partials[1], features0)
